# baseline (speedup 1.0000x reference)
import jax
import jax.numpy as jnp
from jax.experimental import pallas as pl
from jax.experimental.pallas import tpu as pltpu
from jax.experimental.pallas import tpu_sc as plsc

VOCAB = 1000000
EMB = 64
HID = 128
B = 1024
S = 200
N = B * S

GATHER_WINDOW = 128


def _sc_gather(word_emb, flat_ids):
    mesh = plsc.VectorSubcoreMesh(core_axis_name="core", subcore_axis_name="subcore")

    @pl.kernel(
        out_type=jax.ShapeDtypeStruct((N, EMB), word_emb.dtype),
        mesh=mesh,
        compiler_params=pltpu.CompilerParams(use_tc_tiling_on_sc=False),
    )
    def gather_kernel(emb_hbm, ids_hbm, out_hbm):
        def body(ids_vmem, out_vmem):
            pltpu.sync_copy(emb_hbm.at[ids_vmem.at[0]], out_vmem)

        pltpu.emit_pipeline(
            body,
            grid=(N // GATHER_WINDOW,),
            in_specs=[
                pl.BlockSpec((1, GATHER_WINDOW), index_map=lambda i: (0, i))
            ],
            out_specs=[
                pl.BlockSpec((GATHER_WINDOW, EMB), index_map=lambda i: (i, 0))
            ],
            core_axis_name=("core", "subcore"),
            dimension_semantics=(pltpu.PARALLEL,),
        )(ids_hbm, out_hbm)

    return gather_kernel(word_emb, flat_ids)


BBLK = 64


def _tc_fused(we, pe, te, gamma, beta, W, b):
    def tc_kernel(we_ref, pe_ref, te_ref, g_ref, bt_ref, w_ref, b_ref, o_ref):
        x = we_ref[...] + pe_ref[...][None, :, :] + te_ref[0, :][None, None, :]
        mu = jnp.mean(x, axis=-1, keepdims=True)
        xc = x - mu
        var = jnp.mean(xc * xc, axis=-1, keepdims=True)
        xn = xc * jax.lax.rsqrt(var + 1e-12)
        xn = xn * g_ref[0, :][None, None, :] + bt_ref[0, :][None, None, :]
        y = jax.lax.dot_general(
            xn.reshape(BBLK * S, EMB),
            w_ref[...],
            (((1,), (0,)), ((), ())),
            preferred_element_type=jnp.float32,
            precision=jax.lax.Precision.HIGHEST,
        )
        o_ref[...] = (y + b_ref[0, :][None, :]).reshape(BBLK, S, HID)

    return pl.pallas_call(
        tc_kernel,
        grid=(B // BBLK,),
        in_specs=[
            pl.BlockSpec((BBLK, S, EMB), lambda i: (i, 0, 0)),
            pl.BlockSpec((S, EMB), lambda i: (0, 0)),
            pl.BlockSpec((1, EMB), lambda i: (0, 0)),
            pl.BlockSpec((1, EMB), lambda i: (0, 0)),
            pl.BlockSpec((1, EMB), lambda i: (0, 0)),
            pl.BlockSpec((EMB, HID), lambda i: (0, 0)),
            pl.BlockSpec((1, HID), lambda i: (0, 0)),
        ],
        out_specs=pl.BlockSpec((BBLK, S, HID), lambda i: (i, 0, 0)),
        out_shape=jax.ShapeDtypeStruct((B, S, HID), jnp.float32),
    )(we, pe, te, gamma, beta, W, b)


def kernel(input_ids, word_emb, pos_emb, type_emb, ln_gamma, ln_beta, W, b):
    flat_ids = input_ids.astype(jnp.int32).reshape(1, N)
    we = _sc_gather(word_emb, flat_ids).reshape(B, S, EMB)
    pe = pos_emb[:S]
    te = type_emb[0].reshape(1, EMB)
    gamma = ln_gamma.reshape(1, EMB)
    beta = ln_beta.reshape(1, EMB)
    bias = b.reshape(1, HID)
    return _tc_fused(we, pe, te, gamma, beta, W, bias)


# speedup vs baseline: 1.1381x; 1.1381x over previous
import jax
import jax.numpy as jnp
from jax.experimental import pallas as pl
from jax.experimental.pallas import tpu as pltpu
from jax.experimental.pallas import tpu_sc as plsc

VOCAB = 1000000
EMB = 64
EMBP = 128
HID = 128
B = 1024
S = 200
N = B * S

GATHER_WINDOW = 128


def _sc_gather(word_pad, flat_ids):
    mesh = plsc.VectorSubcoreMesh(core_axis_name="core", subcore_axis_name="subcore")

    @pl.kernel(
        out_type=jax.ShapeDtypeStruct((N, EMBP), word_pad.dtype),
        mesh=mesh,
        compiler_params=pltpu.CompilerParams(use_tc_tiling_on_sc=True),
    )
    def gather_kernel(emb_hbm, ids_hbm, out_hbm):
        def body(ids_vmem, out_vmem):
            pltpu.sync_copy(emb_hbm.at[ids_vmem.at[0]], out_vmem)

        pltpu.emit_pipeline(
            body,
            grid=(N // GATHER_WINDOW,),
            in_specs=[
                pl.BlockSpec((1, GATHER_WINDOW), index_map=lambda i: (0, i))
            ],
            out_specs=[
                pl.BlockSpec((GATHER_WINDOW, EMBP), index_map=lambda i: (i, 0))
            ],
            core_axis_name=("core", "subcore"),
            dimension_semantics=(pltpu.PARALLEL,),
        )(ids_hbm, out_hbm)

    return gather_kernel(word_pad, flat_ids)


BBLK = 64


def _tc_fused(we, pe, te, gamma, beta, W, b):
    def tc_kernel(we_ref, pe_ref, te_ref, g_ref, bt_ref, w_ref, b_ref, o_ref):
        x = we_ref[:, :, :EMB] + pe_ref[...][None, :, :] + te_ref[0, :][None, None, :]
        mu = jnp.mean(x, axis=-1, keepdims=True)
        xc = x - mu
        var = jnp.mean(xc * xc, axis=-1, keepdims=True)
        xn = xc * jax.lax.rsqrt(var + 1e-12)
        xn = xn * g_ref[0, :][None, None, :] + bt_ref[0, :][None, None, :]
        y = jax.lax.dot_general(
            xn.reshape(BBLK * S, EMB),
            w_ref[...],
            (((1,), (0,)), ((), ())),
            preferred_element_type=jnp.float32,
            precision=jax.lax.Precision.HIGHEST,
        )
        o_ref[...] = (y + b_ref[0, :][None, :]).reshape(BBLK, S, HID)

    return pl.pallas_call(
        tc_kernel,
        grid=(B // BBLK,),
        in_specs=[
            pl.BlockSpec((BBLK, S, EMBP), lambda i: (i, 0, 0)),
            pl.BlockSpec((S, EMB), lambda i: (0, 0)),
            pl.BlockSpec((1, EMB), lambda i: (0, 0)),
            pl.BlockSpec((1, EMB), lambda i: (0, 0)),
            pl.BlockSpec((1, EMB), lambda i: (0, 0)),
            pl.BlockSpec((EMB, HID), lambda i: (0, 0)),
            pl.BlockSpec((1, HID), lambda i: (0, 0)),
        ],
        out_specs=pl.BlockSpec((BBLK, S, HID), lambda i: (i, 0, 0)),
        out_shape=jax.ShapeDtypeStruct((B, S, HID), jnp.float32),
    )(we, pe, te, gamma, beta, W, b)


def kernel(input_ids, word_emb, pos_emb, type_emb, ln_gamma, ln_beta, W, b):
    flat_ids = input_ids.astype(jnp.int32).reshape(1, N)
    word_pad = jnp.pad(word_emb, ((0, 0), (0, EMBP - EMB)))
    we = _sc_gather(word_pad, flat_ids).reshape(B, S, EMBP)
    pe = pos_emb[:S]
    te = type_emb[0].reshape(1, EMB)
    gamma = ln_gamma.reshape(1, EMB)
    beta = ln_beta.reshape(1, EMB)
    bias = b.reshape(1, HID)
    return _tc_fused(we, pe, te, gamma, beta, W, bias)


# speedup vs baseline: 1.2432x; 1.0923x over previous
import jax
import jax.numpy as jnp
from jax.experimental import pallas as pl
from jax.experimental.pallas import tpu as pltpu
from jax.experimental.pallas import tpu_sc as plsc

VOCAB = 1000000
EMB = 64
EMBP = 128
HID = 128
B = 1024
S = 200
N = B * S

GATHER_WINDOW = 128


def _sc_gather(word_pad, flat_ids):
    mesh = plsc.VectorSubcoreMesh(core_axis_name="core", subcore_axis_name="subcore")

    @pl.kernel(
        out_type=jax.ShapeDtypeStruct((N, EMBP), word_pad.dtype),
        mesh=mesh,
        compiler_params=pltpu.CompilerParams(use_tc_tiling_on_sc=True),
    )
    def gather_kernel(emb_hbm, ids_hbm, out_hbm):
        def body(ids_vmem, out_vmem):
            pltpu.sync_copy(emb_hbm.at[ids_vmem.at[0]], out_vmem)

        pltpu.emit_pipeline(
            body,
            grid=(N // GATHER_WINDOW,),
            in_specs=[
                pl.BlockSpec((1, GATHER_WINDOW), index_map=lambda i: (0, i))
            ],
            out_specs=[
                pl.BlockSpec((GATHER_WINDOW, EMBP), index_map=lambda i: (i, 0))
            ],
            core_axis_name=("core", "subcore"),
            dimension_semantics=(pltpu.PARALLEL,),
        )(ids_hbm, out_hbm)

    return gather_kernel(word_pad, flat_ids)


VBLK = 2048


def _tc_repack(word_t):
    def repack_kernel(wt_ref, o_ref):
        t = jnp.transpose(wt_ref[...], (1, 0))
        o_ref[...] = jnp.concatenate([t, jnp.zeros_like(t)], axis=1)

    return pl.pallas_call(
        repack_kernel,
        grid=(pl.cdiv(VOCAB, VBLK),),
        in_specs=[pl.BlockSpec((EMB, VBLK), lambda i: (0, i))],
        out_specs=pl.BlockSpec((VBLK, EMBP), lambda i: (i, 0)),
        out_shape=jax.ShapeDtypeStruct((VOCAB, EMBP), jnp.float32),
    )(word_t)


BBLK = 64


def _tc_fused(we, pe, te, gamma, beta, W, b):
    def tc_kernel(we_ref, pe_ref, te_ref, g_ref, bt_ref, w_ref, b_ref, o_ref):
        x = we_ref[:, :, :EMB] + pe_ref[...][None, :, :] + te_ref[0, :][None, None, :]
        mu = jnp.mean(x, axis=-1, keepdims=True)
        xc = x - mu
        var = jnp.mean(xc * xc, axis=-1, keepdims=True)
        xn = xc * jax.lax.rsqrt(var + 1e-12)
        xn = xn * g_ref[0, :][None, None, :] + bt_ref[0, :][None, None, :]
        y = jax.lax.dot_general(
            xn.reshape(BBLK * S, EMB),
            w_ref[...],
            (((1,), (0,)), ((), ())),
            preferred_element_type=jnp.float32,
            precision=jax.lax.Precision.HIGHEST,
        )
        o_ref[...] = (y + b_ref[0, :][None, :]).reshape(BBLK, S, HID)

    return pl.pallas_call(
        tc_kernel,
        grid=(B // BBLK,),
        in_specs=[
            pl.BlockSpec((BBLK, S, EMBP), lambda i: (i, 0, 0)),
            pl.BlockSpec((S, EMB), lambda i: (0, 0)),
            pl.BlockSpec((1, EMB), lambda i: (0, 0)),
            pl.BlockSpec((1, EMB), lambda i: (0, 0)),
            pl.BlockSpec((1, EMB), lambda i: (0, 0)),
            pl.BlockSpec((EMB, HID), lambda i: (0, 0)),
            pl.BlockSpec((1, HID), lambda i: (0, 0)),
        ],
        out_specs=pl.BlockSpec((BBLK, S, HID), lambda i: (i, 0, 0)),
        out_shape=jax.ShapeDtypeStruct((B, S, HID), jnp.float32),
    )(we, pe, te, gamma, beta, W, b)


def kernel(input_ids, word_emb, pos_emb, type_emb, ln_gamma, ln_beta, W, b):
    flat_ids = input_ids.astype(jnp.int32).reshape(1, N)
    word_pad = _tc_repack(word_emb.T)
    we = _sc_gather(word_pad, flat_ids).reshape(B, S, EMBP)
    pe = pos_emb[:S]
    te = type_emb[0].reshape(1, EMB)
    gamma = ln_gamma.reshape(1, EMB)
    beta = ln_beta.reshape(1, EMB)
    bias = b.reshape(1, HID)
    return _tc_fused(we, pe, te, gamma, beta, W, bias)


# speedup vs baseline: 1.3016x; 1.0470x over previous
import jax
import jax.numpy as jnp
from jax.experimental import pallas as pl
from jax.experimental.pallas import tpu as pltpu
from jax.experimental.pallas import tpu_sc as plsc

VOCAB = 1000000
EMB = 64
EMBP = 128
HID = 128
B = 1024
S = 200
N = B * S

GATHER_WINDOW = 128


def _sc_gather(word_pad, flat_ids):
    mesh = plsc.VectorSubcoreMesh(core_axis_name="core", subcore_axis_name="subcore")

    @pl.kernel(
        out_type=jax.ShapeDtypeStruct((N, EMBP), word_pad.dtype),
        mesh=mesh,
        compiler_params=pltpu.CompilerParams(use_tc_tiling_on_sc=True),
    )
    def gather_kernel(emb_hbm, ids_hbm, out_hbm):
        def body(ids_vmem, out_vmem):
            pltpu.sync_copy(emb_hbm.at[ids_vmem.at[0]], out_vmem)

        pltpu.emit_pipeline(
            body,
            grid=(N // GATHER_WINDOW,),
            in_specs=[
                pl.BlockSpec((1, GATHER_WINDOW), index_map=lambda i: (0, i))
            ],
            out_specs=[
                pl.BlockSpec((GATHER_WINDOW, EMBP), index_map=lambda i: (i, 0))
            ],
            core_axis_name=("core", "subcore"),
            dimension_semantics=(pltpu.PARALLEL,),
        )(ids_hbm, out_hbm)

    return gather_kernel(word_pad, flat_ids)


VBLK = 4096


def _tc_repack(word_t):
    def repack_kernel(wt_ref, o_ref):
        o_ref[:, :EMB] = jnp.transpose(wt_ref[...], (1, 0))

    return pl.pallas_call(
        repack_kernel,
        grid=(pl.cdiv(VOCAB, VBLK),),
        in_specs=[pl.BlockSpec((EMB, VBLK), lambda i: (0, i))],
        out_specs=pl.BlockSpec((VBLK, EMBP), lambda i: (i, 0)),
        out_shape=jax.ShapeDtypeStruct((VOCAB, EMBP), jnp.float32),
    )(word_t)


BBLK = 32


def _tc_fused(we, pe, te, gamma, beta, W, b):
    def tc_kernel(we_ref, pe_ref, te_ref, g_ref, bt_ref, w_ref, b_ref, o_ref):
        x = we_ref[:, :, :EMB] + pe_ref[...][None, :, :] + te_ref[0, :][None, None, :]
        xf = x.reshape(BBLK * S, EMB)
        ones = jnp.full((EMB, 1), 1.0 / EMB, dtype=jnp.float32)
        mu = jax.lax.dot_general(
            xf, ones, (((1,), (0,)), ((), ())),
            preferred_element_type=jnp.float32,
            precision=jax.lax.Precision.HIGHEST,
        )
        msq = jax.lax.dot_general(
            xf * xf, ones, (((1,), (0,)), ((), ())),
            preferred_element_type=jnp.float32,
            precision=jax.lax.Precision.HIGHEST,
        )
        var = msq - mu * mu
        rs = jax.lax.rsqrt(var + 1e-12)
        xc = (xf - mu).astype(jnp.bfloat16)
        wp = (w_ref[...] * g_ref[0, :][:, None]).astype(jnp.bfloat16)
        y = jax.lax.dot_general(
            xc, wp, (((1,), (0,)), ((), ())),
            preferred_element_type=jnp.float32,
        )
        bp = jax.lax.dot_general(
            bt_ref[...], w_ref[...], (((1,), (0,)), ((), ())),
            preferred_element_type=jnp.float32,
            precision=jax.lax.Precision.HIGHEST,
        ) + b_ref[...]
        o_ref[...] = (y * rs + bp).reshape(BBLK, S, HID)

    return pl.pallas_call(
        tc_kernel,
        grid=(B // BBLK,),
        in_specs=[
            pl.BlockSpec((BBLK, S, EMBP), lambda i: (i, 0, 0)),
            pl.BlockSpec((S, EMB), lambda i: (0, 0)),
            pl.BlockSpec((1, EMB), lambda i: (0, 0)),
            pl.BlockSpec((1, EMB), lambda i: (0, 0)),
            pl.BlockSpec((1, EMB), lambda i: (0, 0)),
            pl.BlockSpec((EMB, HID), lambda i: (0, 0)),
            pl.BlockSpec((1, HID), lambda i: (0, 0)),
        ],
        out_specs=pl.BlockSpec((BBLK, S, HID), lambda i: (i, 0, 0)),
        out_shape=jax.ShapeDtypeStruct((B, S, HID), jnp.float32),
    )(we, pe, te, gamma, beta, W, b)


def kernel(input_ids, word_emb, pos_emb, type_emb, ln_gamma, ln_beta, W, b):
    flat_ids = input_ids.astype(jnp.int32).reshape(1, N)
    word_pad = _tc_repack(word_emb.T)
    we = _sc_gather(word_pad, flat_ids).reshape(B, S, EMBP)
    pe = pos_emb[:S]
    te = type_emb[0].reshape(1, EMB)
    gamma = ln_gamma.reshape(1, EMB)
    beta = ln_beta.reshape(1, EMB)
    bias = b.reshape(1, HID)
    return _tc_fused(we, pe, te, gamma, beta, W, bias)


# speedup vs baseline: 1.9772x; 1.5190x over previous
import jax
import jax.numpy as jnp
from jax.experimental import pallas as pl
from jax.experimental.pallas import tpu as pltpu
from jax.experimental.pallas import tpu_sc as plsc

VOCAB = 1000000
EMB = 64
EMBP = 128
HID = 128
B = 1024
S = 200
N = B * S

GATHER_WINDOW = 128


def _sc_gather(word_pad, flat_ids):
    mesh = plsc.VectorSubcoreMesh(core_axis_name="core", subcore_axis_name="subcore")

    @pl.kernel(
        out_type=jax.ShapeDtypeStruct((N, EMBP), word_pad.dtype),
        mesh=mesh,
        compiler_params=pltpu.CompilerParams(use_tc_tiling_on_sc=True),
    )
    def gather_kernel(emb_hbm, ids_hbm, out_hbm):
        def body(ids_vmem, out_vmem):
            pltpu.sync_copy(emb_hbm.at[ids_vmem.at[0]], out_vmem)

        pltpu.emit_pipeline(
            body,
            grid=(N // GATHER_WINDOW,),
            in_specs=[
                pl.BlockSpec((1, GATHER_WINDOW), index_map=lambda i: (0, i))
            ],
            out_specs=[
                pl.BlockSpec((GATHER_WINDOW, EMBP), index_map=lambda i: (i, 0))
            ],
            core_axis_name=("core", "subcore"),
            dimension_semantics=(pltpu.PARALLEL,),
        )(ids_hbm, out_hbm)

    return gather_kernel(word_pad, flat_ids)


VBLK = 16384


def _tc_repack(word_t):
    def repack_kernel(wt_ref, o_ref):
        o_ref[:, :EMB] = jnp.transpose(wt_ref[...], (1, 0))

    return pl.pallas_call(
        repack_kernel,
        grid=(pl.cdiv(VOCAB, VBLK),),
        in_specs=[pl.BlockSpec((EMB, VBLK), lambda i: (0, i))],
        out_specs=pl.BlockSpec((VBLK, EMBP), lambda i: (i, 0)),
        out_shape=jax.ShapeDtypeStruct((VOCAB, EMBP), jnp.float32),
    )(word_t)


RBLK = 12800


def _tc_fused(we, pet, gamma, beta, W, b):
    def tc_kernel(we_ref, pet_ref, g_ref, bt_ref, w_ref, b_ref, o_ref):
        lane = jax.lax.broadcasted_iota(jnp.int32, (RBLK, EMBP), 1)
        x = jnp.where(lane < EMB, we_ref[...] + pet_ref[...], 0.0)
        cmu = jnp.where(
            jax.lax.broadcasted_iota(jnp.int32, (EMBP, 1), 0) < EMB,
            1.0 / EMB, 0.0)
        mu = jax.lax.dot_general(
            x, cmu, (((1,), (0,)), ((), ())),
            preferred_element_type=jnp.float32,
        )
        msq = jax.lax.dot_general(
            x * x, cmu, (((1,), (0,)), ((), ())),
            preferred_element_type=jnp.float32,
        )
        var = msq - mu * mu
        rs = jax.lax.rsqrt(var + 1e-12)
        xc = (x - mu).astype(jnp.bfloat16)
        rowi = jax.lax.broadcasted_iota(jnp.int32, (EMBP, HID), 0)
        wp = jnp.where(rowi < EMB, w_ref[...] * g_ref[0, :][:, None], 0.0)
        y = jax.lax.dot_general(
            xc, wp.astype(jnp.bfloat16), (((1,), (0,)), ((), ())),
            preferred_element_type=jnp.float32,
        )
        bp = jax.lax.dot_general(
            bt_ref[...], w_ref[...], (((1,), (0,)), ((), ())),
            preferred_element_type=jnp.float32,
        ) + b_ref[...]
        o_ref[...] = y * rs + bp

    return pl.pallas_call(
        tc_kernel,
        grid=(N // RBLK,),
        in_specs=[
            pl.BlockSpec((RBLK, EMBP), lambda i: (i, 0)),
            pl.BlockSpec((RBLK, EMBP), lambda i: (0, 0)),
            pl.BlockSpec((1, EMBP), lambda i: (0, 0)),
            pl.BlockSpec((1, EMBP), lambda i: (0, 0)),
            pl.BlockSpec((EMBP, HID), lambda i: (0, 0)),
            pl.BlockSpec((1, HID), lambda i: (0, 0)),
        ],
        out_specs=pl.BlockSpec((RBLK, HID), lambda i: (i, 0)),
        out_shape=jax.ShapeDtypeStruct((N, HID), jnp.float32),
    )(we, pet, gamma, beta, W, b)


def kernel(input_ids, word_emb, pos_emb, type_emb, ln_gamma, ln_beta, W, b):
    flat_ids = input_ids.astype(jnp.int32).reshape(1, N)
    word_pad = _tc_repack(word_emb.T)
    we = _sc_gather(word_pad, flat_ids)
    pet = jnp.tile(pos_emb[:S] + type_emb[0][None, :], (RBLK // S, 1))
    pet = jnp.pad(pet, ((0, 0), (0, EMBP - EMB)))
    gamma = jnp.pad(ln_gamma.reshape(1, EMB), ((0, 0), (0, EMBP - EMB)))
    beta = jnp.pad(ln_beta.reshape(1, EMB), ((0, 0), (0, EMBP - EMB)))
    Wp = jnp.pad(W, ((0, EMBP - EMB), (0, 0)))
    bias = b.reshape(1, HID)
    out = _tc_fused(we, pet, gamma, beta, Wp, bias)
    return out.reshape(B, S, HID)


# speedup vs baseline: 2.0786x; 1.0513x over previous
import jax
import jax.numpy as jnp
from jax.experimental import pallas as pl
from jax.experimental.pallas import tpu as pltpu
from jax.experimental.pallas import tpu_sc as plsc

VOCAB = 1000000
EMB = 64
EMBP = 128
HID = 128
B = 1024
S = 200
N = B * S

GATHER_WINDOW = 256


def _sc_gather(word_pad, flat_ids):
    mesh = plsc.VectorSubcoreMesh(core_axis_name="core", subcore_axis_name="subcore")

    @pl.kernel(
        out_type=jax.ShapeDtypeStruct((N, EMBP), word_pad.dtype),
        mesh=mesh,
        compiler_params=pltpu.CompilerParams(use_tc_tiling_on_sc=True),
    )
    def gather_kernel(emb_hbm, ids_hbm, out_hbm):
        def body(ids_vmem, out_vmem):
            pltpu.sync_copy(emb_hbm.at[ids_vmem.at[0]], out_vmem)

        pltpu.emit_pipeline(
            body,
            grid=(N // GATHER_WINDOW,),
            in_specs=[
                pl.BlockSpec((1, GATHER_WINDOW), index_map=lambda i: (0, i))
            ],
            out_specs=[
                pl.BlockSpec((GATHER_WINDOW, EMBP), index_map=lambda i: (i, 0))
            ],
            core_axis_name=("core", "subcore"),
            dimension_semantics=(pltpu.PARALLEL,),
        )(ids_hbm, out_hbm)

    return gather_kernel(word_pad, flat_ids)


VBLK = 32768


def _tc_repack(word_t):
    def repack_kernel(wt_ref, o_ref):
        o_ref[:, :EMB] = jnp.transpose(wt_ref[...], (1, 0))

    return pl.pallas_call(
        repack_kernel,
        grid=(pl.cdiv(VOCAB, VBLK),),
        in_specs=[pl.BlockSpec((EMB, VBLK), lambda i: (0, i))],
        out_specs=pl.BlockSpec((VBLK, EMBP), lambda i: (i, 0)),
        out_shape=jax.ShapeDtypeStruct((VOCAB, EMBP), jnp.float32),
    )(word_t)


RBLK = 12800


def _tc_fused(we, pet, gamma, beta, W, b):
    def tc_kernel(we_ref, pet_ref, g_ref, bt_ref, w_ref, b_ref, o_ref):
        lane = jax.lax.broadcasted_iota(jnp.int32, (RBLK, EMBP), 1)
        x = jnp.where(lane < EMB, we_ref[...] + pet_ref[...], 0.0)
        cmu = jnp.where(
            jax.lax.broadcasted_iota(jnp.int32, (EMBP, 1), 0) < EMB,
            1.0 / EMB, 0.0)
        mu = jax.lax.dot_general(
            x, cmu, (((1,), (0,)), ((), ())),
            preferred_element_type=jnp.float32,
        )
        msq = jax.lax.dot_general(
            x * x, cmu, (((1,), (0,)), ((), ())),
            preferred_element_type=jnp.float32,
        )
        var = msq - mu * mu
        rs = jax.lax.rsqrt(var + 1e-12)
        xc = (x - mu).astype(jnp.bfloat16)
        rowi = jax.lax.broadcasted_iota(jnp.int32, (EMBP, HID), 0)
        wp = jnp.where(rowi < EMB, w_ref[...] * g_ref[0, :][:, None], 0.0)
        y = jax.lax.dot_general(
            xc, wp.astype(jnp.bfloat16), (((1,), (0,)), ((), ())),
            preferred_element_type=jnp.float32,
        )
        bp = jax.lax.dot_general(
            bt_ref[...], w_ref[...], (((1,), (0,)), ((), ())),
            preferred_element_type=jnp.float32,
        ) + b_ref[...]
        o_ref[...] = y * rs + bp

    return pl.pallas_call(
        tc_kernel,
        grid=(N // RBLK,),
        in_specs=[
            pl.BlockSpec((RBLK, EMBP), lambda i: (i, 0)),
            pl.BlockSpec((RBLK, EMBP), lambda i: (0, 0)),
            pl.BlockSpec((1, EMBP), lambda i: (0, 0)),
            pl.BlockSpec((1, EMBP), lambda i: (0, 0)),
            pl.BlockSpec((EMBP, HID), lambda i: (0, 0)),
            pl.BlockSpec((1, HID), lambda i: (0, 0)),
        ],
        out_specs=pl.BlockSpec((RBLK, HID), lambda i: (i, 0)),
        out_shape=jax.ShapeDtypeStruct((N, HID), jnp.float32),
    )(we, pet, gamma, beta, W, b)


def kernel(input_ids, word_emb, pos_emb, type_emb, ln_gamma, ln_beta, W, b):
    flat_ids = input_ids.astype(jnp.int32).reshape(1, N)
    word_pad = _tc_repack(word_emb.T)
    we = _sc_gather(word_pad, flat_ids)
    pet = jnp.tile(pos_emb[:S] + type_emb[0][None, :], (RBLK // S, 1))
    pet = jnp.pad(pet, ((0, 0), (0, EMBP - EMB)))
    gamma = jnp.pad(ln_gamma.reshape(1, EMB), ((0, 0), (0, EMBP - EMB)))
    beta = jnp.pad(ln_beta.reshape(1, EMB), ((0, 0), (0, EMBP - EMB)))
    Wp = jnp.pad(W, ((0, EMBP - EMB), (0, 0)))
    bias = b.reshape(1, HID)
    out = _tc_fused(we, pet, gamma, beta, Wp, bias)
    return out.reshape(B, S, HID)


# speedup vs baseline: 2.0908x; 1.0059x over previous
import jax
import jax.numpy as jnp
from jax.experimental import pallas as pl
from jax.experimental.pallas import tpu as pltpu
from jax.experimental.pallas import tpu_sc as plsc

VOCAB = 1000000
EMB = 64
EMBP = 128
HID = 128
B = 1024
S = 200
N = B * S

GATHER_WINDOW = 256


NCHUNK = 4
NC = N // NCHUNK


def _sc_gather(word_pad, flat_ids):
    mesh = plsc.VectorSubcoreMesh(core_axis_name="core", subcore_axis_name="subcore")

    @pl.kernel(
        out_type=jax.ShapeDtypeStruct((NC, EMBP), word_pad.dtype),
        mesh=mesh,
        compiler_params=pltpu.CompilerParams(use_tc_tiling_on_sc=True),
    )
    def gather_kernel(emb_hbm, ids_hbm, out_hbm):
        def body(ids_vmem, out_vmem):
            pltpu.sync_copy(emb_hbm.at[ids_vmem.at[0]], out_vmem)

        pltpu.emit_pipeline(
            body,
            grid=(NC // GATHER_WINDOW,),
            in_specs=[
                pl.BlockSpec((1, GATHER_WINDOW), index_map=lambda i: (0, i))
            ],
            out_specs=[
                pl.BlockSpec((GATHER_WINDOW, EMBP), index_map=lambda i: (i, 0))
            ],
            core_axis_name=("core", "subcore"),
            dimension_semantics=(pltpu.PARALLEL,),
        )(ids_hbm, out_hbm)

    return gather_kernel(word_pad, flat_ids)


VBLK = 32768


def _tc_repack(word_t):
    def repack_kernel(wt_ref, o_ref):
        o_ref[:, :EMB] = jnp.transpose(wt_ref[...], (1, 0))

    return pl.pallas_call(
        repack_kernel,
        grid=(pl.cdiv(VOCAB, VBLK),),
        in_specs=[pl.BlockSpec((EMB, VBLK), lambda i: (0, i))],
        out_specs=pl.BlockSpec((VBLK, EMBP), lambda i: (i, 0)),
        out_shape=jax.ShapeDtypeStruct((VOCAB, EMBP), jnp.float32),
    )(word_t)


RBLK = 12800


def _tc_fused(we, pet, gamma, beta, W, b, acc, chunk):
    def tc_kernel(we_ref, pet_ref, g_ref, bt_ref, w_ref, b_ref, *acc_and_o):
        o_ref = acc_and_o[-1]
        lane = jax.lax.broadcasted_iota(jnp.int32, (RBLK, EMBP), 1)
        x = jnp.where(lane < EMB, we_ref[...] + pet_ref[...], 0.0)
        cmu = jnp.where(
            jax.lax.broadcasted_iota(jnp.int32, (EMBP, 1), 0) < EMB,
            1.0 / EMB, 0.0)
        mu = jax.lax.dot_general(
            x, cmu, (((1,), (0,)), ((), ())),
            preferred_element_type=jnp.float32,
        )
        msq = jax.lax.dot_general(
            x * x, cmu, (((1,), (0,)), ((), ())),
            preferred_element_type=jnp.float32,
        )
        var = msq - mu * mu
        rs = jax.lax.rsqrt(var + 1e-12)
        xc = (x - mu).astype(jnp.bfloat16)
        rowi = jax.lax.broadcasted_iota(jnp.int32, (EMBP, HID), 0)
        wp = jnp.where(rowi < EMB, w_ref[...] * g_ref[0, :][:, None], 0.0)
        y = jax.lax.dot_general(
            xc, wp.astype(jnp.bfloat16), (((1,), (0,)), ((), ())),
            preferred_element_type=jnp.float32,
        )
        bp = jax.lax.dot_general(
            bt_ref[...], w_ref[...], (((1,), (0,)), ((), ())),
            preferred_element_type=jnp.float32,
        ) + b_ref[...]
        o_ref[...] = y * rs + bp

    base = chunk * (NC // RBLK)
    in_specs = [
        pl.BlockSpec((RBLK, EMBP), lambda i: (i, 0)),
        pl.BlockSpec((RBLK, EMBP), lambda i: (0, 0)),
        pl.BlockSpec((1, EMBP), lambda i: (0, 0)),
        pl.BlockSpec((1, EMBP), lambda i: (0, 0)),
        pl.BlockSpec((EMBP, HID), lambda i: (0, 0)),
        pl.BlockSpec((1, HID), lambda i: (0, 0)),
    ]
    args = [we, pet, gamma, beta, W, b]
    aliases = {}
    if acc is not None:
        in_specs.append(pl.BlockSpec(memory_space=pl.ANY))
        args.append(acc)
        aliases = {6: 0}
    return pl.pallas_call(
        tc_kernel,
        grid=(NC // RBLK,),
        in_specs=in_specs,
        out_specs=pl.BlockSpec((RBLK, HID), lambda i: (base + i, 0)),
        out_shape=jax.ShapeDtypeStruct((N, HID), jnp.float32),
        input_output_aliases=aliases,
    )(*args)


def kernel(input_ids, word_emb, pos_emb, type_emb, ln_gamma, ln_beta, W, b):
    flat_ids = input_ids.astype(jnp.int32).reshape(1, N)
    word_pad = _tc_repack(word_emb.T)
    pet = jnp.tile(pos_emb[:S] + type_emb[0][None, :], (RBLK // S, 1))
    pet = jnp.pad(pet, ((0, 0), (0, EMBP - EMB)))
    gamma = jnp.pad(ln_gamma.reshape(1, EMB), ((0, 0), (0, EMBP - EMB)))
    beta = jnp.pad(ln_beta.reshape(1, EMB), ((0, 0), (0, EMBP - EMB)))
    Wp = jnp.pad(W, ((0, EMBP - EMB), (0, 0)))
    bias = b.reshape(1, HID)
    gathered = [
        _sc_gather(word_pad, jax.lax.dynamic_slice(flat_ids, (0, c * NC), (1, NC)))
        for c in range(NCHUNK)
    ]
    out = None
    for c in range(NCHUNK):
        out = _tc_fused(gathered[c], pet, gamma, beta, Wp, bias, out, c)
    return out.reshape(B, S, HID)


# speedup vs baseline: 2.1094x; 1.0089x over previous
import jax
import jax.numpy as jnp
from jax.experimental import pallas as pl
from jax.experimental.pallas import tpu as pltpu
from jax.experimental.pallas import tpu_sc as plsc

VOCAB = 1000000
EMB = 64
EMBP = 128
HID = 128
B = 1024
S = 200
N = B * S

GATHER_WINDOW = 256

NCHUNK = 2
NC = N // NCHUNK


def _sc_gather(word_pad, flat_ids):
    mesh = plsc.VectorSubcoreMesh(core_axis_name="core", subcore_axis_name="subcore")

    @pl.kernel(
        out_type=jax.ShapeDtypeStruct((NC, EMBP), word_pad.dtype),
        mesh=mesh,
        compiler_params=pltpu.CompilerParams(use_tc_tiling_on_sc=True),
    )
    def gather_kernel(emb_hbm, ids_hbm, out_hbm):
        def body(ids_vmem, out_vmem):
            pltpu.sync_copy(emb_hbm.at[ids_vmem.at[0]], out_vmem)

        pltpu.emit_pipeline(
            body,
            grid=(NC // GATHER_WINDOW,),
            in_specs=[
                pl.BlockSpec((1, GATHER_WINDOW), index_map=lambda i: (0, i))
            ],
            out_specs=[
                pl.BlockSpec((GATHER_WINDOW, EMBP), index_map=lambda i: (i, 0))
            ],
            core_axis_name=("core", "subcore"),
            dimension_semantics=(pltpu.PARALLEL,),
        )(ids_hbm, out_hbm)

    return gather_kernel(word_pad, flat_ids)


VBLK = 32768


def _tc_repack(word_t):
    def repack_kernel(wt_ref, o_ref):
        o_ref[:, :EMB] = jnp.transpose(wt_ref[...], (1, 0))

    return pl.pallas_call(
        repack_kernel,
        grid=(pl.cdiv(VOCAB, VBLK),),
        in_specs=[pl.BlockSpec((EMB, VBLK), lambda i: (0, i))],
        out_specs=pl.BlockSpec((VBLK, EMBP), lambda i: (i, 0)),
        out_shape=jax.ShapeDtypeStruct((VOCAB, EMBP), jnp.float32),
    )(word_t)


RBLK = 12800


def _tc_fused(we, pet, gamma, beta, W, b, acc, chunk):
    def tc_kernel(we_ref, pet_ref, g_ref, bt_ref, w_ref, b_ref, *acc_and_o):
        o_ref = acc_and_o[-1]
        lane = jax.lax.broadcasted_iota(jnp.int32, (RBLK, EMBP), 1)
        x = jnp.where(lane < EMB, we_ref[...] + pet_ref[...], 0.0)
        cmu = jnp.where(
            jax.lax.broadcasted_iota(jnp.int32, (EMBP, 1), 0) < EMB,
            1.0 / EMB, 0.0)
        mu = jax.lax.dot_general(
            x, cmu, (((1,), (0,)), ((), ())),
            preferred_element_type=jnp.float32,
        )
        msq = jax.lax.dot_general(
            x * x, cmu, (((1,), (0,)), ((), ())),
            preferred_element_type=jnp.float32,
        )
        var = msq - mu * mu
        rs = jax.lax.rsqrt(var + 1e-12)
        xc = (x - mu).astype(jnp.bfloat16)
        rowi = jax.lax.broadcasted_iota(jnp.int32, (EMBP, HID), 0)
        wp = jnp.where(rowi < EMB, w_ref[...] * g_ref[0, :][:, None], 0.0)
        y = jax.lax.dot_general(
            xc, wp.astype(jnp.bfloat16), (((1,), (0,)), ((), ())),
            preferred_element_type=jnp.float32,
        )
        bp = jax.lax.dot_general(
            bt_ref[...], w_ref[...], (((1,), (0,)), ((), ())),
            preferred_element_type=jnp.float32,
        ) + b_ref[...]
        o_ref[...] = y * rs + bp

    base = chunk * (NC // RBLK)
    in_specs = [
        pl.BlockSpec((RBLK, EMBP), lambda i: (i, 0)),
        pl.BlockSpec((RBLK, EMBP), lambda i: (0, 0)),
        pl.BlockSpec((1, EMBP), lambda i: (0, 0)),
        pl.BlockSpec((1, EMBP), lambda i: (0, 0)),
        pl.BlockSpec((EMBP, HID), lambda i: (0, 0)),
        pl.BlockSpec((1, HID), lambda i: (0, 0)),
    ]
    args = [we, pet, gamma, beta, W, b]
    aliases = {}
    if acc is not None:
        in_specs.append(pl.BlockSpec(memory_space=pl.ANY))
        args.append(acc)
        aliases = {6: 0}
    return pl.pallas_call(
        tc_kernel,
        grid=(NC // RBLK,),
        in_specs=in_specs,
        out_specs=pl.BlockSpec((RBLK, HID), lambda i: (base + i, 0)),
        out_shape=jax.ShapeDtypeStruct((N, HID), jnp.float32),
        input_output_aliases=aliases,
    )(*args)


def kernel(input_ids, word_emb, pos_emb, type_emb, ln_gamma, ln_beta, W, b):
    flat_ids = input_ids.astype(jnp.int32).reshape(1, N)
    word_pad = _tc_repack(word_emb.T)
    pet = jnp.tile(pos_emb[:S] + type_emb[0][None, :], (RBLK // S, 1))
    pet = jnp.pad(pet, ((0, 0), (0, EMBP - EMB)))
    gamma = jnp.pad(ln_gamma.reshape(1, EMB), ((0, 0), (0, EMBP - EMB)))
    beta = jnp.pad(ln_beta.reshape(1, EMB), ((0, 0), (0, EMBP - EMB)))
    Wp = jnp.pad(W, ((0, EMBP - EMB), (0, 0)))
    bias = b.reshape(1, HID)
    gathered = [
        _sc_gather(word_pad, jax.lax.dynamic_slice(flat_ids, (0, c * NC), (1, NC)))
        for c in range(NCHUNK)
    ]
    out = None
    for c in range(NCHUNK):
        out = _tc_fused(gathered[c], pet, gamma, beta, Wp, bias, out, c)
    return out.reshape(B, S, HID)
